# Initial kernel scaffold; baseline (speedup 1.0000x reference)
import jax
import jax.numpy as jnp
from jax.experimental import pallas as pl


def kernel(input, from_id, to_id, W, b):
    raise NotImplementedError("write your pallas kernel here")


# speedup vs baseline: 4.3006x; 4.3006x over previous
import jax
import jax.numpy as jnp
from jax import lax
from jax.experimental import pallas as pl
from jax.experimental.pallas import tpu as pltpu
from jax.experimental.pallas import tpu_sc as plsc

N_NODES = 100000
N_HALO = 400000
D = 100
DP = 128

NC = 2
NS = 16
NW = NC * NS

PAIRS_PER_W = 12544
PAD_TOTAL = NW * PAIRS_PER_W

WIN_BITS = 13
WIN = 1 << WIN_BITS
NWIN = 13
PAD_FROM = NWIN * WIN
LAST_ROWS = N_NODES - (NWIN - 1) * WIN
LPT = 112
SEG_CAP = 12544
BLK = 128

SUPER = 2048
RING = 4096
TRASH = NWIN * RING

BR = 1000


def _mm_body(x_ref, wt_ref, b_ref, o_ref):
    o_ref[...] = (
        jnp.dot(x_ref[...], wt_ref[...], preferred_element_type=jnp.float32)
        + b_ref[...]
    )


def _matmul(x, wt, b2):
    return pl.pallas_call(
        _mm_body,
        grid=(N_NODES // BR,),
        in_specs=[
            pl.BlockSpec((BR, D), lambda i: (i, 0)),
            pl.BlockSpec((D, DP), lambda i: (0, 0)),
            pl.BlockSpec((1, DP), lambda i: (0, 0)),
        ],
        out_specs=pl.BlockSpec((BR, DP), lambda i: (i, 0)),
        out_shape=jax.ShapeDtypeStruct((N_NODES, DP), jnp.float32),
    )(x, wt, b2)


_SUPERS = (128, 128, 128, 128, 128, 128, 16)
assert sum(_SUPERS) * 16 == PAIRS_PER_W


ZSLOT = TRASH + 16


def _partition_body(f_hbm, t_hbm, fseg_hbm, tseg_hbm, counts_hbm,
                    fstage, tstage, fbufs, tbufs, cvbuf):
    c = lax.axis_index("c")
    s = lax.axis_index("s")
    wid = s * NC + c
    base = wid * PAIRS_PER_W
    lane = lax.iota(jnp.int32, 16)

    def make_vec_body(nvec):
        def vec_body(i, carry):
            offs = carry
            iv = jnp.minimum(i, nvec - 1)
            f = fstage[pl.ds(iv * 16, 16)]
            t = tstage[pl.ds(iv * 16, 16)]
            bkt = lax.shift_right_logical(f, WIN_BITS) + jnp.where(
                i < nvec, jnp.int32(0), jnp.int32(64))
            floc = lax.bitwise_and(f, WIN - 1)
            new_offs = []
            for b in range(NWIN):
                m = bkt == b
                cs = plsc.cumsum(jnp.where(m, jnp.int32(1), jnp.int32(0)))
                wr = lax.bitwise_and(offs[b] + cs - 1, RING - 1)
                dest = jnp.where(m, b * RING + wr, TRASH + lane)
                plsc.store_scatter(fbufs, [dest], floc)
                plsc.store_scatter(tbufs, [dest], t)
                new_offs.append(offs[b] + cs[15])
            return tuple(new_offs)
        return vec_body

    offs = tuple(jnp.int32(0) for _ in range(NWIN))
    flushed = tuple(jnp.int32(0) for _ in range(NWIN))
    vec0 = 0
    for step, nvec in enumerate(_SUPERS):
        pltpu.sync_copy(f_hbm.at[pl.ds(base + vec0 * 16, nvec * 16)],
                        fstage.at[pl.ds(0, nvec * 16)])
        pltpu.sync_copy(t_hbm.at[pl.ds(base + vec0 * 16, nvec * 16)],
                        tstage.at[pl.ds(0, nvec * 16)])
        offs = lax.fori_loop(0, nvec + 1, make_vec_body(nvec), offs)
        vec0 += nvec
        last = step == len(_SUPERS) - 1

        if last:
            new_offs = []
            for b in range(NWIN):
                off = offs[b]
                dummy_f = WIN + lax.bitwise_and(lane, 3)
                dummy_t = lax.bitwise_and(
                    lane * 619 + wid * 97 + b * 1031, jnp.int32(0xFFFF))
                tgt = lax.bitwise_and(off + 127, jnp.int32(-128))

                def pad_body(jj, _, b=b, off=off, df=dummy_f, dt=dummy_t):
                    idx = b * RING + lax.bitwise_and(
                        off + jj * 16 + lane, RING - 1)
                    plsc.store_scatter(fbufs, [idx], df)
                    plsc.store_scatter(tbufs, [idx], dt)
                    return 0

                lax.fori_loop(0, lax.shift_right_logical(tgt - off + 15, 4),
                              pad_body, 0)
                new_offs.append(tgt)
            offs = tuple(new_offs)

        new_flushed = []
        for b in range(NWIN):
            off, fl = offs[b], flushed[b]
            limit = lax.bitwise_and(off, jnp.int32(-128))

            def flush_body(q, _, b=b, fl=fl):
                pos = lax.bitwise_and(fl + q * BLK, RING - 1)
                src = pl.ds(pl.multiple_of(b * RING + pos, BLK), BLK)
                dst = pl.ds(pl.multiple_of(fl + q * BLK, BLK), BLK)
                pltpu.sync_copy(fbufs.at[src], fseg_hbm.at[b, wid, dst])
                pltpu.sync_copy(tbufs.at[src], tseg_hbm.at[b, wid, dst])
                return 0

            lax.fori_loop(0, lax.shift_right_logical(limit - fl, 7),
                          flush_body, 0)
            new_flushed.append(limit)
        flushed = tuple(new_flushed)

    cv = jnp.zeros((16,), jnp.int32)
    for b in range(NWIN):
        cv = jnp.where(lane == b, flushed[b], cv)
    cvbuf[...] = cv
    pltpu.sync_copy(cvbuf, counts_hbm.at[wid])


_partition = pl.kernel(
    _partition_body,
    out_type=(
        jax.ShapeDtypeStruct((NWIN, NW, SEG_CAP), jnp.int32),
        jax.ShapeDtypeStruct((NWIN, NW, SEG_CAP), jnp.int32),
        jax.ShapeDtypeStruct((NW, 16), jnp.int32),
    ),
    mesh=plsc.VectorSubcoreMesh(core_axis_name="c", subcore_axis_name="s"),
    compiler_params=pltpu.CompilerParams(needs_layout_passes=False),
    scratch_types=[
        pltpu.VMEM((SUPER,), jnp.int32),
        pltpu.VMEM((SUPER,), jnp.int32),
        pltpu.VMEM((NWIN * RING + 16,), jnp.int32),
        pltpu.VMEM((NWIN * RING + 16,), jnp.int32),
        pltpu.VMEM((16,), jnp.int32),
    ],
)

_FIRST = (0, 6)
_NWIN_C = (6, 7)


def _scatter_body(a_hbm, fseg_hbm, tseg_hbm, counts_hbm, out_hbm,
                  fidx, tidx, rows, cbuf, window, sem):
    c = lax.axis_index("c")
    s = lax.axis_index("s")
    lane = lax.iota(jnp.int32, 16)
    first = jnp.where(c == 0, _FIRST[0], _FIRST[1])
    nwin_c = jnp.where(c == 0, _NWIN_C[0], _NWIN_C[1])

    pltpu.sync_copy(counts_hbm, cbuf)

    for j in range(max(_NWIN_C)):
        w = first + j

        @pl.when(j < nwin_c)
        def _():
            row0 = pl.multiple_of(w * WIN, WIN)

            def _win_io(to_window):
                def do(tb, n):
                    tbm = pl.multiple_of(tb, 8)
                    rb = pl.multiple_of(row0 + tbm, 8)
                    if to_window:
                        pltpu.sync_copy(a_hbm.at[pl.ds(rb, n)],
                                        window.at[pl.ds(tbm, n)])
                    else:
                        pltpu.sync_copy(window.at[pl.ds(tbm, n)],
                                        out_hbm.at[pl.ds(rb, n)])

                @pl.when(w != NWIN - 1)
                def _():
                    do(s * (WIN // NS), WIN // NS)

                @pl.when(w == NWIN - 1)
                def _():
                    @pl.when(s < NS - 1)
                    def _():
                        do(s * LPT, LPT)

                    @pl.when(s == NS - 1)
                    def _():
                        do((NS - 1) * LPT, LAST_ROWS - (NS - 1) * LPT)

            _win_io(True)

            plsc.subcore_barrier()

            for segi in range(NW // NS):
                seg = s + segi * NS
                cv = cbuf[seg]
                tgt = jnp.max(jnp.where(lane == w, cv, 0))
                nblk = lax.shift_right_logical(tgt, 7)

                def blk_body(kk, _, w=w, seg=seg):
                    pltpu.sync_copy(
                        fseg_hbm.at[w, seg, pl.ds(kk * BLK, BLK)], fidx)
                    pltpu.sync_copy(
                        tseg_hbm.at[w, seg, pl.ds(kk * BLK, BLK)], tidx)
                    pltpu.async_copy(a_hbm.at[tidx], rows, sem).wait()
                    pltpu.sync_copy(rows, window.at[fidx], add=True)
                    return 0

                lax.fori_loop(0, nblk, blk_body, 0)

            plsc.subcore_barrier()

            _win_io(False)

            plsc.subcore_barrier()


_scatter = pl.kernel(
    _scatter_body,
    out_type=jax.ShapeDtypeStruct((N_NODES, DP), jnp.float32),
    mesh=plsc.VectorSubcoreMesh(core_axis_name="c", subcore_axis_name="s"),
    compiler_params=pltpu.CompilerParams(needs_layout_passes=False),
    scratch_types=[
        pltpu.VMEM((BLK,), jnp.int32),
        pltpu.VMEM((BLK,), jnp.int32),
        pltpu.VMEM((BLK, DP), jnp.float32),
        pltpu.VMEM((NW, 16), jnp.int32),
        pltpu.VMEM_SHARED((WIN + 4, DP), jnp.float32),
        pltpu.SemaphoreType.DMA,
    ],
)


@jax.jit
def kernel(input, from_id, to_id, W, b):
    wt = jnp.pad(W.T.astype(jnp.float32), ((0, 0), (0, DP - D)))
    b2 = jnp.pad(b.astype(jnp.float32), (0, DP - D)).reshape(1, DP)
    a = _matmul(input, wt, b2)

    f32i = from_id.astype(jnp.int32)
    t32i = to_id.astype(jnp.int32)
    npad = PAD_TOTAL - N_HALO
    fpad = jnp.concatenate(
        [f32i, jnp.full((npad,), PAD_FROM, jnp.int32)])
    tpad = jnp.concatenate([t32i, jnp.zeros((npad,), jnp.int32)])

    fseg, tseg, counts = _partition(fpad, tpad)
    out_pad = _scatter(a, fseg, tseg, counts)
    return out_pad[:, :D]


# speedup vs baseline: 5.3746x; 1.2497x over previous
import jax
import jax.numpy as jnp
from jax import lax
from jax.experimental import pallas as pl
from jax.experimental.pallas import tpu as pltpu
from jax.experimental.pallas import tpu_sc as plsc

N_NODES = 100000
N_HALO = 400000
D = 100
DP = 128

NC = 2
NS = 16
NW = NC * NS

PAIRS_PER_W = 12544
PAD_TOTAL = NW * PAIRS_PER_W

WIN_BITS = 13
WIN = 1 << WIN_BITS
NWIN = 13
PAD_FROM = NWIN * WIN
LAST_ROWS = N_NODES - (NWIN - 1) * WIN
LPT = 112
SEG_CAP = 12544
BLK = 128

SUPER = 2048
RING = 4096
TRASH = NWIN * RING

BR = 1000


def _mm_body(x_ref, wt_ref, b_ref, o_ref):
    o_ref[...] = (
        jnp.dot(x_ref[...], wt_ref[...], preferred_element_type=jnp.float32)
        + b_ref[...]
    )


def _matmul(x, wt, b2):
    return pl.pallas_call(
        _mm_body,
        grid=(N_NODES // BR,),
        in_specs=[
            pl.BlockSpec((BR, D), lambda i: (i, 0)),
            pl.BlockSpec((D, DP), lambda i: (0, 0)),
            pl.BlockSpec((1, DP), lambda i: (0, 0)),
        ],
        out_specs=pl.BlockSpec((BR, DP), lambda i: (i, 0)),
        out_shape=jax.ShapeDtypeStruct((N_NODES, DP), jnp.float32),
    )(x, wt, b2)


_SUPERS = (128, 128, 128, 128, 128, 128, 16)
assert sum(_SUPERS) * 16 == PAIRS_PER_W


ZSLOT = TRASH + 16


def _partition_body(f_hbm, t_hbm, fseg_hbm, tseg_hbm, counts_hbm,
                    fstage, tstage, fbufs, tbufs, cvbuf):
    c = lax.axis_index("c")
    s = lax.axis_index("s")
    wid = s * NC + c
    base = wid * PAIRS_PER_W
    lane = lax.iota(jnp.int32, 16)

    def make_vec_body(nvec):
        def vec_body(i, carry):
            offs = carry
            iv = jnp.minimum(i, nvec - 1)
            f = fstage[pl.ds(iv * 16, 16)]
            t = tstage[pl.ds(iv * 16, 16)]
            bkt = lax.shift_right_logical(f, WIN_BITS) + jnp.where(
                i < nvec, jnp.int32(0), jnp.int32(64))
            floc = lax.bitwise_and(f, WIN - 1)
            new_offs = []
            for b in range(NWIN):
                m = bkt == b
                cs = plsc.cumsum(jnp.where(m, jnp.int32(1), jnp.int32(0)))
                wr = lax.bitwise_and(offs[b] + cs - 1, RING - 1)
                dest = jnp.where(m, b * RING + wr, TRASH + lane)
                plsc.store_scatter(fbufs, [dest], floc)
                plsc.store_scatter(tbufs, [dest], t)
                new_offs.append(offs[b] + cs[15])
            return tuple(new_offs)
        return vec_body

    offs = tuple(jnp.int32(0) for _ in range(NWIN))
    flushed = tuple(jnp.int32(0) for _ in range(NWIN))
    vec0 = 0
    for step, nvec in enumerate(_SUPERS):
        pltpu.sync_copy(f_hbm.at[pl.ds(base + vec0 * 16, nvec * 16)],
                        fstage.at[pl.ds(0, nvec * 16)])
        pltpu.sync_copy(t_hbm.at[pl.ds(base + vec0 * 16, nvec * 16)],
                        tstage.at[pl.ds(0, nvec * 16)])
        offs = lax.fori_loop(0, nvec + 1, make_vec_body(nvec), offs)
        vec0 += nvec
        last = step == len(_SUPERS) - 1

        if last:
            new_offs = []
            for b in range(NWIN):
                off = offs[b]
                dummy_f = WIN + lax.bitwise_and(lane, 3)
                dummy_t = lax.bitwise_and(
                    lane * 619 + wid * 97 + b * 1031, jnp.int32(0xFFFF))
                tgt = lax.bitwise_and(off + 127, jnp.int32(-128))

                def pad_body(jj, _, b=b, off=off, df=dummy_f, dt=dummy_t):
                    idx = b * RING + lax.bitwise_and(
                        off + jj * 16 + lane, RING - 1)
                    plsc.store_scatter(fbufs, [idx], df)
                    plsc.store_scatter(tbufs, [idx], dt)
                    return 0

                lax.fori_loop(0, lax.shift_right_logical(tgt - off + 15, 4),
                              pad_body, 0)
                new_offs.append(tgt)
            offs = tuple(new_offs)

        new_flushed = []
        for b in range(NWIN):
            off, fl = offs[b], flushed[b]
            limit = lax.bitwise_and(off, jnp.int32(-128))

            def flush_body(q, _, b=b, fl=fl):
                pos = lax.bitwise_and(fl + q * BLK, RING - 1)
                src = pl.ds(pl.multiple_of(b * RING + pos, BLK), BLK)
                row = lax.shift_right_logical(fl, 7) + q
                pltpu.sync_copy(fbufs.at[src], fseg_hbm.at[b, wid, row])
                pltpu.sync_copy(tbufs.at[src], tseg_hbm.at[b, wid, row])
                return 0

            lax.fori_loop(0, lax.shift_right_logical(limit - fl, 7),
                          flush_body, 0)
            new_flushed.append(limit)
        flushed = tuple(new_flushed)

    cv = jnp.zeros((16,), jnp.int32)
    for b in range(NWIN):
        cv = jnp.where(lane == b, flushed[b], cv)
    cvbuf[...] = cv
    pltpu.sync_copy(cvbuf, counts_hbm.at[wid])


_partition = pl.kernel(
    _partition_body,
    out_type=(
        jax.ShapeDtypeStruct((NWIN, NW, SEG_CAP // BLK, BLK), jnp.int32),
        jax.ShapeDtypeStruct((NWIN, NW, SEG_CAP // BLK, BLK), jnp.int32),
        jax.ShapeDtypeStruct((NW, 16), jnp.int32),
    ),
    mesh=plsc.VectorSubcoreMesh(core_axis_name="c", subcore_axis_name="s"),
    compiler_params=pltpu.CompilerParams(needs_layout_passes=False),
    scratch_types=[
        pltpu.VMEM((SUPER,), jnp.int32),
        pltpu.VMEM((SUPER,), jnp.int32),
        pltpu.VMEM((NWIN * RING + 16,), jnp.int32),
        pltpu.VMEM((NWIN * RING + 16,), jnp.int32),
        pltpu.VMEM((16,), jnp.int32),
    ],
)

_FIRST = (0, 6)
_NWIN_C = (6, 7)


def _scatter_body(a_hbm, fseg_hbm, tseg_hbm, counts_hbm, out_hbm,
                  fblk, tblk, rows, cbuf, window, sem):
    c = lax.axis_index("c")
    s = lax.axis_index("s")
    lane = lax.iota(jnp.int32, 16)
    first = jnp.where(c == 0, _FIRST[0], _FIRST[1])
    nwin_c = jnp.where(c == 0, _NWIN_C[0], _NWIN_C[1])

    pltpu.sync_copy(counts_hbm, cbuf)

    for j in range(max(_NWIN_C)):
        w = first + j

        @pl.when(j < nwin_c)
        def _():
            row0 = pl.multiple_of(w * WIN, WIN)

            def _win_io(to_window):
                def do(tb, n):
                    tbm = pl.multiple_of(tb, 8)
                    rb = pl.multiple_of(row0 + tbm, 8)
                    if to_window:
                        pltpu.sync_copy(a_hbm.at[pl.ds(rb, n)],
                                        window.at[pl.ds(tbm, n)])
                    else:
                        pltpu.sync_copy(window.at[pl.ds(tbm, n)],
                                        out_hbm.at[pl.ds(rb, n)])

                @pl.when(w != NWIN - 1)
                def _():
                    do(s * (WIN // NS), WIN // NS)

                @pl.when(w == NWIN - 1)
                def _():
                    @pl.when(s < NS - 1)
                    def _():
                        do(s * LPT, LPT)

                    @pl.when(s == NS - 1)
                    def _():
                        do((NS - 1) * LPT, LAST_ROWS - (NS - 1) * LPT)

            _win_io(True)

            plsc.subcore_barrier()

            for segi in range(NW // NS):
                seg = s + segi * NS
                cv = cbuf[seg]
                tgt = jnp.max(jnp.where(lane == w, cv, 0))
                nblk = lax.shift_right_logical(tgt, 7)

                def ld_body(q, _, w=w, seg=seg):
                    pltpu.sync_copy(fseg_hbm.at[w, seg, pl.ds(q * 2, 2)],
                                    fblk.at[pl.ds(q * 2, 2)])
                    pltpu.sync_copy(tseg_hbm.at[w, seg, pl.ds(q * 2, 2)],
                                    tblk.at[pl.ds(q * 2, 2)])
                    return 0

                lax.fori_loop(0, lax.shift_right_logical(nblk + 1, 1),
                              ld_body, 0)

                def blk_body(kk, _):
                    pb = lax.bitwise_and(kk, 1)
                    pltpu.async_copy(a_hbm.at[tblk.at[kk]], rows.at[pb],
                                     sem.at[pb])

                    @pl.when(kk > 0)
                    def _():
                        qb = 1 - pb
                        pltpu.make_async_copy(
                            a_hbm.at[tblk.at[kk - 1]], rows.at[qb],
                            sem.at[qb]).wait()
                        pltpu.sync_copy(rows.at[qb],
                                        window.at[fblk.at[kk - 1]],
                                        add=True)
                    return 0

                lax.fori_loop(0, nblk, blk_body, 0)

                @pl.when(nblk > 0)
                def _():
                    kk = nblk - 1
                    pb = lax.bitwise_and(kk, 1)
                    pltpu.make_async_copy(
                        a_hbm.at[tblk.at[kk]], rows.at[pb],
                        sem.at[pb]).wait()
                    pltpu.sync_copy(rows.at[pb], window.at[fblk.at[kk]],
                                    add=True)

            plsc.subcore_barrier()

            _win_io(False)

            plsc.subcore_barrier()


_scatter = pl.kernel(
    _scatter_body,
    out_type=jax.ShapeDtypeStruct((N_NODES, DP), jnp.float32),
    mesh=plsc.VectorSubcoreMesh(core_axis_name="c", subcore_axis_name="s"),
    compiler_params=pltpu.CompilerParams(needs_layout_passes=False),
    scratch_types=[
        pltpu.VMEM((SEG_CAP // BLK, BLK), jnp.int32),
        pltpu.VMEM((SEG_CAP // BLK, BLK), jnp.int32),
        pltpu.VMEM((2, BLK, DP), jnp.float32),
        pltpu.VMEM((NW, 16), jnp.int32),
        pltpu.VMEM_SHARED((WIN + 4, DP), jnp.float32),
        pltpu.SemaphoreType.DMA((2,)),
    ],
)


@jax.jit
def kernel(input, from_id, to_id, W, b):
    wt = jnp.pad(W.T.astype(jnp.float32), ((0, 0), (0, DP - D)))
    b2 = jnp.pad(b.astype(jnp.float32), (0, DP - D)).reshape(1, DP)
    a = _matmul(input, wt, b2)

    f32i = from_id.astype(jnp.int32)
    t32i = to_id.astype(jnp.int32)
    npad = PAD_TOTAL - N_HALO
    fpad = jnp.concatenate(
        [f32i, jnp.full((npad,), PAD_FROM, jnp.int32)])
    tpad = jnp.concatenate([t32i, jnp.zeros((npad,), jnp.int32)])

    fseg, tseg, counts = _partition(fpad, tpad)
    out_pad = _scatter(a, fseg, tseg, counts)
    return out_pad[:, :D]


# speedup vs baseline: 5.6190x; 1.0455x over previous
import jax
import jax.numpy as jnp
from jax import lax
from jax.experimental import pallas as pl
from jax.experimental.pallas import tpu as pltpu
from jax.experimental.pallas import tpu_sc as plsc

N_NODES = 100000
N_HALO = 400000
D = 100
DP = 128

NC = 2
NS = 16
NW = NC * NS

PAIRS_PER_W = 12544
PAD_TOTAL = NW * PAIRS_PER_W

WIN_BITS = 13
WIN = 1 << WIN_BITS
NWIN = 13
PAD_FROM = NWIN * WIN
LAST_ROWS = N_NODES - (NWIN - 1) * WIN
LPT = 112
SEG_CAP = 12544
SEG_R = 100
BLK = 128

SUPER = 2048
RING = 4096
TRASH = NWIN * RING

BR = 1000


def _mm_body(x_ref, wt_ref, b_ref, o_ref):
    o_ref[...] = (
        jnp.dot(x_ref[...], wt_ref[...], preferred_element_type=jnp.float32)
        + b_ref[...]
    )


def _matmul(x, wt, b2):
    return pl.pallas_call(
        _mm_body,
        grid=(N_NODES // BR,),
        in_specs=[
            pl.BlockSpec((BR, D), lambda i: (i, 0)),
            pl.BlockSpec((D, DP), lambda i: (0, 0)),
            pl.BlockSpec((1, DP), lambda i: (0, 0)),
        ],
        out_specs=pl.BlockSpec((BR, DP), lambda i: (i, 0)),
        out_shape=jax.ShapeDtypeStruct((N_NODES, DP), jnp.float32),
    )(x, wt, b2)


_SUPERS = (128, 128, 128, 128, 128, 128, 16)
assert sum(_SUPERS) * 16 == PAIRS_PER_W


ZSLOT = TRASH + 16


def _partition_body(f_hbm, t_hbm, fseg_hbm, tseg_hbm, counts_hbm,
                    fstage, tstage, fbufs, tbufs, cvbuf):
    c = lax.axis_index("c")
    s = lax.axis_index("s")
    wid = s * NC + c
    base = wid * PAIRS_PER_W
    lane = lax.iota(jnp.int32, 16)

    def make_vec_body(nvec):
        def vec_body(i, carry):
            offs = carry
            iv = jnp.minimum(i, nvec - 1)
            f = fstage[pl.ds(iv * 16, 16)]
            t = tstage[pl.ds(iv * 16, 16)]
            bkt = lax.shift_right_logical(f, WIN_BITS) + jnp.where(
                i < nvec, jnp.int32(0), jnp.int32(64))
            floc = lax.bitwise_and(f, WIN - 1)
            new_offs = []
            for b in range(NWIN):
                m = bkt == b
                cs = plsc.cumsum(jnp.where(m, jnp.int32(1), jnp.int32(0)))
                wr = lax.bitwise_and(offs[b] + cs - 1, RING - 1)
                dest = jnp.where(m, b * RING + wr, TRASH + lane)
                plsc.store_scatter(fbufs, [dest], floc)
                plsc.store_scatter(tbufs, [dest], t)
                new_offs.append(offs[b] + cs[15])
            return tuple(new_offs)
        return vec_body

    offs = tuple(jnp.int32(0) for _ in range(NWIN))
    flushed = tuple(jnp.int32(0) for _ in range(NWIN))
    vec0 = 0
    for step, nvec in enumerate(_SUPERS):
        pltpu.sync_copy(f_hbm.at[pl.ds(base + vec0 * 16, nvec * 16)],
                        fstage.at[pl.ds(0, nvec * 16)])
        pltpu.sync_copy(t_hbm.at[pl.ds(base + vec0 * 16, nvec * 16)],
                        tstage.at[pl.ds(0, nvec * 16)])
        offs = lax.fori_loop(0, nvec + 1, make_vec_body(nvec), offs)
        vec0 += nvec
        last = step == len(_SUPERS) - 1

        if last:
            new_offs = []
            for b in range(NWIN):
                off = offs[b]
                dummy_f = WIN + lax.bitwise_and(lane, 3)
                dummy_t = lax.bitwise_and(
                    lane * 619 + wid * 97 + b * 1031, jnp.int32(0xFFFF))
                tgt = lax.bitwise_and(off + 127, jnp.int32(-128))

                def pad_body(jj, _, b=b, off=off, df=dummy_f, dt=dummy_t):
                    idx = b * RING + lax.bitwise_and(
                        off + jj * 16 + lane, RING - 1)
                    plsc.store_scatter(fbufs, [idx], df)
                    plsc.store_scatter(tbufs, [idx], dt)
                    return 0

                lax.fori_loop(0, lax.shift_right_logical(tgt - off + 15, 4),
                              pad_body, 0)
                new_offs.append(tgt)
            offs = tuple(new_offs)

        new_flushed = []
        for b in range(NWIN):
            off, fl = offs[b], flushed[b]
            limit = lax.bitwise_and(off, jnp.int32(-128))

            def flush_body(q, _, b=b, fl=fl):
                pos = lax.bitwise_and(fl + q * BLK, RING - 1)
                src = pl.ds(pl.multiple_of(b * RING + pos, BLK), BLK)
                row = lax.shift_right_logical(fl, 7) + q
                pltpu.sync_copy(fbufs.at[src], fseg_hbm.at[b, wid, row])
                pltpu.sync_copy(tbufs.at[src], tseg_hbm.at[b, wid, row])
                return 0

            lax.fori_loop(0, lax.shift_right_logical(limit - fl, 7),
                          flush_body, 0)
            new_flushed.append(limit)
        flushed = tuple(new_flushed)

    cv = jnp.zeros((16,), jnp.int32)
    for b in range(NWIN):
        cv = jnp.where(lane == b, flushed[b], cv)
    cvbuf[...] = cv
    pltpu.sync_copy(cvbuf, counts_hbm.at[wid])


_partition = pl.kernel(
    _partition_body,
    out_type=(
        jax.ShapeDtypeStruct((NWIN, NW, SEG_R, BLK), jnp.int32),
        jax.ShapeDtypeStruct((NWIN, NW, SEG_R, BLK), jnp.int32),
        jax.ShapeDtypeStruct((NW, 16), jnp.int32),
    ),
    mesh=plsc.VectorSubcoreMesh(core_axis_name="c", subcore_axis_name="s"),
    compiler_params=pltpu.CompilerParams(needs_layout_passes=False),
    scratch_types=[
        pltpu.VMEM((SUPER,), jnp.int32),
        pltpu.VMEM((SUPER,), jnp.int32),
        pltpu.VMEM((NWIN * RING + 16,), jnp.int32),
        pltpu.VMEM((NWIN * RING + 16,), jnp.int32),
        pltpu.VMEM((16,), jnp.int32),
    ],
)

_FIRST = (0, 6)
_NWIN_C = (6, 7)


def _scatter_body(a_hbm, fseg_hbm, tseg_hbm, counts_hbm, out_hbm,
                  fblk, tblk, rows, cbuf, window, sem):
    c = lax.axis_index("c")
    s = lax.axis_index("s")
    lane = lax.iota(jnp.int32, 16)
    first = jnp.where(c == 0, _FIRST[0], _FIRST[1])
    nwin_c = jnp.where(c == 0, _NWIN_C[0], _NWIN_C[1])

    pltpu.sync_copy(counts_hbm, cbuf)

    for j in range(max(_NWIN_C)):
        w = first + j

        @pl.when(j < nwin_c)
        def _():
            row0 = pl.multiple_of(w * WIN, WIN)

            def _win_io(to_window):
                def do(tb, n):
                    tbm = pl.multiple_of(tb, 8)
                    rb = pl.multiple_of(row0 + tbm, 8)
                    if to_window:
                        pltpu.sync_copy(a_hbm.at[pl.ds(rb, n)],
                                        window.at[pl.ds(tbm, n)])
                    else:
                        pltpu.sync_copy(window.at[pl.ds(tbm, n)],
                                        out_hbm.at[pl.ds(rb, n)])

                @pl.when(w != NWIN - 1)
                def _():
                    do(s * (WIN // NS), WIN // NS)

                @pl.when(w == NWIN - 1)
                def _():
                    @pl.when(s < NS - 1)
                    def _():
                        do(s * LPT, LPT)

                    @pl.when(s == NS - 1)
                    def _():
                        do((NS - 1) * LPT, LAST_ROWS - (NS - 1) * LPT)

            _win_io(True)

            plsc.subcore_barrier()

            for segi in range(NW // NS):
                seg = s + segi * NS
                cv = cbuf[seg]
                tgt = jnp.max(jnp.where(lane == w, cv, 0))
                nblk = lax.shift_right_logical(tgt, 7)

                def ld_body(q, _, w=w, seg=seg):
                    pltpu.sync_copy(fseg_hbm.at[w, seg, pl.ds(q * 4, 4)],
                                    fblk.at[pl.ds(q * 4, 4)])
                    pltpu.sync_copy(tseg_hbm.at[w, seg, pl.ds(q * 4, 4)],
                                    tblk.at[pl.ds(q * 4, 4)])
                    return 0

                lax.fori_loop(0, lax.shift_right_logical(nblk + 3, 2),
                              ld_body, 0)

                def blk_body(kk, _):
                    pb = lax.bitwise_and(kk, 1)
                    pltpu.async_copy(a_hbm.at[tblk.at[kk]], rows.at[pb],
                                     sem.at[pb])

                    @pl.when(kk > 0)
                    def _():
                        qb = 1 - pb
                        pltpu.make_async_copy(
                            a_hbm.at[tblk.at[kk - 1]], rows.at[qb],
                            sem.at[qb]).wait()
                        pltpu.sync_copy(rows.at[qb],
                                        window.at[fblk.at[kk - 1]],
                                        add=True)
                    return 0

                lax.fori_loop(0, nblk, blk_body, 0)

                @pl.when(nblk > 0)
                def _():
                    kk = nblk - 1
                    pb = lax.bitwise_and(kk, 1)
                    pltpu.make_async_copy(
                        a_hbm.at[tblk.at[kk]], rows.at[pb],
                        sem.at[pb]).wait()
                    pltpu.sync_copy(rows.at[pb], window.at[fblk.at[kk]],
                                    add=True)

            plsc.subcore_barrier()

            _win_io(False)

            plsc.subcore_barrier()


_scatter = pl.kernel(
    _scatter_body,
    out_type=jax.ShapeDtypeStruct((N_NODES, DP), jnp.float32),
    mesh=plsc.VectorSubcoreMesh(core_axis_name="c", subcore_axis_name="s"),
    compiler_params=pltpu.CompilerParams(needs_layout_passes=False),
    scratch_types=[
        pltpu.VMEM((SEG_R, BLK), jnp.int32),
        pltpu.VMEM((SEG_R, BLK), jnp.int32),
        pltpu.VMEM((2, BLK, DP), jnp.float32),
        pltpu.VMEM((NW, 16), jnp.int32),
        pltpu.VMEM_SHARED((WIN + 4, DP), jnp.float32),
        pltpu.SemaphoreType.DMA((2,)),
    ],
)


@jax.jit
def kernel(input, from_id, to_id, W, b):
    wt = jnp.pad(W.T.astype(jnp.float32), ((0, 0), (0, DP - D)))
    b2 = jnp.pad(b.astype(jnp.float32), (0, DP - D)).reshape(1, DP)
    a = _matmul(input, wt, b2)

    f32i = from_id.astype(jnp.int32)
    t32i = to_id.astype(jnp.int32)
    npad = PAD_TOTAL - N_HALO
    fpad = jnp.concatenate(
        [f32i, jnp.full((npad,), PAD_FROM, jnp.int32)])
    tpad = jnp.concatenate([t32i, jnp.zeros((npad,), jnp.int32)])

    fseg, tseg, counts = _partition(fpad, tpad)
    out_pad = _scatter(a, fseg, tseg, counts)
    return out_pad[:, :D]
